# Initial kernel scaffold; baseline (speedup 1.0000x reference)
import jax
import jax.numpy as jnp
from jax.experimental import pallas as pl


def kernel(inputs, table):
    raise NotImplementedError("write your pallas kernel here")


# speedup vs baseline: 3.6465x; 3.6465x over previous
import functools

import jax
import jax.numpy as jnp
from jax import lax
from jax.experimental import pallas as pl
from jax.experimental.pallas import tpu as pltpu
from jax.experimental.pallas import tpu_sc as plsc

_INFO = plsc.get_sparse_core_info()
_NC = _INFO.num_cores
_NS = _INFO.num_subcores
_NW = _NC * _NS

_ROWS = 8192
_D = 1024
_BSZ = 4
_ROWS_PER_W = _ROWS // _NW
_CHUNK = 64
_NCHUNK = _ROWS_PER_W // _CHUNK


def _body(table_hbm, out_hbm, buf):
    wid = lax.axis_index("s") * _NC + lax.axis_index("c")
    base = wid * _ROWS_PER_W
    for i in range(_NCHUNK):
        r0 = base + i * _CHUNK
        pltpu.sync_copy(table_hbm.at[pl.ds(r0, _CHUNK), :], buf)
        for b in range(_BSZ):
            pltpu.sync_copy(buf, out_hbm.at[b, pl.ds(r0, _CHUNK), :])


@jax.jit
def _broadcast_table(table):
    mesh = plsc.VectorSubcoreMesh(core_axis_name="c", subcore_axis_name="s")
    return pl.kernel(
        _body,
        out_type=jax.ShapeDtypeStruct((_BSZ, _ROWS, _D), jnp.float32),
        mesh=mesh,
        scratch_types=[pltpu.VMEM((_CHUNK, _D), jnp.float32)],
    )(table)


def kernel(inputs, table):
    return _broadcast_table(table)
